# baseline (speedup 1.0000x reference)
import jax
import jax.numpy as jnp
from jax import lax
from jax.experimental import pallas as pl
from jax.experimental.pallas import tpu as pltpu
from jax.experimental.pallas import tpu_sc as plsc

_B = 4
_N = 2048
_RPT = 256
_G = 8
_HALVES = _RPT // (16 * _G)


def _sqrt16(x):
    x = jnp.maximum(x, jnp.float32(1e-24))
    i = lax.bitcast_convert_type(x, jnp.int32)
    i = jnp.int32(0x5F3759DF) - (i >> 1)
    y = lax.bitcast_convert_type(i, jnp.float32)
    h = jnp.float32(0.5) * x
    for _ in range(3):
        y = y * (jnp.float32(1.5) - h * y * y)
    return x * y


def _round_bf16(x):
    i = lax.bitcast_convert_type(x, jnp.int32)
    i = i + jnp.int32(0x7FFF) + ((i >> 16) & jnp.int32(1))
    i = i & jnp.int32(-65536)
    return lax.bitcast_convert_type(i, jnp.float32)


def _tree_min(vs):
    while len(vs) > 1:
        vs = [jnp.minimum(vs[i], vs[i + 1]) for i in range(0, len(vs) - 1, 2)] \
            + ([vs[-1]] if len(vs) % 2 else [])
    return vs[0]


_INF16 = lambda: jnp.full((16,), jnp.float32(jnp.inf))


def _sc_body(a1_hbm, a2_hbm, out_hbm,
             xv, yv, yr, y2b, xr, x2b, colp, tmp8, accv, shared):
    cid = lax.axis_index("c")
    sid = lax.axis_index("s")
    b = cid * 2 + sid // 8
    slot = sid % 8
    r0 = slot * _RPT

    pltpu.sync_copy(a1_hbm.at[b], xv)
    pltpu.sync_copy(a2_hbm.at[b], yv)

    def prep_y(k, carry):
        s = k * 16
        v0 = yv[0, pl.ds(s, 16)]
        v1 = yv[1, pl.ds(s, 16)]
        v2 = yv[2, pl.ds(s, 16)]
        yr[0, pl.ds(s, 16)] = _round_bf16(v0)
        yr[1, pl.ds(s, 16)] = _round_bf16(v1)
        yr[2, pl.ds(s, 16)] = _round_bf16(v2)
        y2b[pl.ds(s, 16)] = v0 * v0 + v1 * v1 + v2 * v2
        colp[pl.ds(s, 16)] = _INF16()
        return carry

    lax.fori_loop(0, _N // 16, prep_y, 0)

    def prep_x(k, carry):
        s = k * 16
        v0 = xv[0, pl.ds(r0 + s, 16)]
        v1 = xv[1, pl.ds(r0 + s, 16)]
        v2 = xv[2, pl.ds(r0 + s, 16)]
        xr[0, pl.ds(s, 16)] = _round_bf16(v0)
        xr[1, pl.ds(s, 16)] = _round_bf16(v1)
        xr[2, pl.ds(s, 16)] = _round_bf16(v2)
        x2b[pl.ds(s, 16)] = v0 * v0 + v1 * v1 + v2 * v2
        return carry

    lax.fori_loop(0, _RPT // 16, prep_x, 0)

    lanes = lax.iota(jnp.int32, 16)

    rowsum = jnp.zeros((16,), jnp.float32)
    for half in range(_HALVES):
        xbase = half * 16 * _G
        xs = []
        x2s = []
        for gi in range(_G):
            off = xbase + gi * 16
            xs.append((xr[0, pl.ds(off, 16)],
                       xr[1, pl.ds(off, 16)],
                       xr[2, pl.ds(off, 16)]))
            x2s.append(x2b[pl.ds(off, 16)])

        def mbody(mc, accs, xs=xs, x2s=x2s):
            s = mc * 16
            o0 = yr[0, pl.ds(s, 16)]
            o1 = yr[1, pl.ds(s, 16)]
            o2 = yr[2, pl.ds(s, 16)]
            oy2 = y2b[pl.ds(s, 16)]
            accs = list(accs)
            colv = _INF16()
            for j in range(16):
                b0 = o0[j] + o0[j]
                b1 = o1[j] + o1[j]
                b2 = o2[j] + o2[j]
                y2s = oy2[j]
                parts = []
                for gi in range(_G):
                    q0, q1, q2 = xs[gi]
                    t = q0 * b0 + q1 * b1 + q2 * b2
                    dd = (x2s[gi] + y2s) - t
                    dd = jnp.maximum(dd, jnp.float32(0.0))
                    accs[gi] = jnp.minimum(accs[gi], dd)
                    parts.append(dd)
                p = _tree_min(parts)
                sc = _tree_min([p[l] for l in range(16)])
                colv = jnp.where(lanes == jnp.int32(j), sc, colv)
            colp[pl.ds(s, 16)] = jnp.minimum(colp[pl.ds(s, 16)], colv)
            return tuple(accs)

        accs = lax.fori_loop(0, _N // 16, mbody,
                             tuple(_INF16() for _ in range(_G)))
        for gi in range(_G):
            rowsum = rowsum + _sqrt16(accs[gi])

    pltpu.sync_copy(colp, shared.at[sid])
    plsc.subcore_barrier()
    sbase = (sid // 8) * 8
    m0 = slot * _RPT
    pltpu.sync_copy(shared.at[pl.ds(sbase, 8), pl.ds(m0, _RPT)], tmp8)

    colsum = jnp.zeros((16,), jnp.float32)
    for k in range(_RPT // 16):
        vs = [tmp8[p, pl.ds(k * 16, 16)] for p in range(8)]
        colsum = colsum + _sqrt16(_tree_min(vs))

    accv[0, :] = rowsum
    accv[1, :] = colsum
    pltpu.sync_copy(accv, out_hbm.at[cid * 16 + sid])


@jax.jit
def _sc_chamfer(a1t, a2t):
    mesh = plsc.VectorSubcoreMesh(core_axis_name="c", subcore_axis_name="s")
    run = pl.kernel(
        _sc_body,
        out_type=jax.ShapeDtypeStruct((32, 2, 16), jnp.float32),
        mesh=mesh,
        scratch_types=[
            pltpu.VMEM((3, _N), jnp.float32),
            pltpu.VMEM((3, _N), jnp.float32),
            pltpu.VMEM((3, _N), jnp.float32),
            pltpu.VMEM((_N,), jnp.float32),
            pltpu.VMEM((3, _RPT), jnp.float32),
            pltpu.VMEM((_RPT,), jnp.float32),
            pltpu.VMEM((_N,), jnp.float32),
            pltpu.VMEM((8, _RPT), jnp.float32),
            pltpu.VMEM((2, 16), jnp.float32),
            pltpu.VMEM_SHARED((16, _N), jnp.float32),
        ],
    )
    return run(a1t, a2t)


def kernel(array1, array2):
    a1t = jnp.transpose(array1, (0, 2, 1))
    a2t = jnp.transpose(array2, (0, 2, 1))
    parts = _sc_chamfer(a1t, a2t)
    total = jnp.sum(parts)
    return total / jnp.float32(2 * _B * _N)


# speedup vs baseline: 2.2491x; 2.2491x over previous
import jax
import jax.numpy as jnp
from jax import lax
from jax.experimental import pallas as pl
from jax.experimental.pallas import tpu as pltpu
from jax.experimental.pallas import tpu_sc as plsc

_B = 4
_N = 2048
_RPT = 256
_G = 8
_HALVES = _RPT // (16 * _G)


def _sqrt16(x):
    x = jnp.maximum(x, jnp.float32(1e-24))
    i = lax.bitcast_convert_type(x, jnp.int32)
    i = jnp.int32(0x5F3759DF) - (i >> 1)
    y = lax.bitcast_convert_type(i, jnp.float32)
    h = jnp.float32(0.5) * x
    for _ in range(3):
        y = y * (jnp.float32(1.5) - h * y * y)
    return x * y


def _round_bf16(x):
    i = lax.bitcast_convert_type(x, jnp.int32)
    i = i + jnp.int32(0x7FFF) + ((i >> 16) & jnp.int32(1))
    i = i & jnp.int32(-65536)
    return lax.bitcast_convert_type(i, jnp.float32)


def _take16(v, idx):
    return lax.gather(
        v, idx[:, None],
        lax.GatherDimensionNumbers(offset_dims=(), collapsed_slice_dims=(0,),
                                   start_index_map=(0,)),
        (1,), mode=lax.GatherScatterMode.PROMISE_IN_BOUNDS)


def _tree_min(vs):
    while len(vs) > 1:
        vs = [jnp.minimum(vs[i], vs[i + 1]) for i in range(0, len(vs) - 1, 2)] \
            + ([vs[-1]] if len(vs) % 2 else [])
    return vs[0]


_INF16 = lambda: jnp.full((16,), jnp.float32(jnp.inf))


def _sc_body(a1_hbm, a2_hbm, out_hbm,
             xv, yv, yr, y2b, xr, x2b, colp, tmp8, accv, shared):
    cid = lax.axis_index("c")
    sid = lax.axis_index("s")
    b = cid * 2 + sid // 8
    slot = sid % 8
    r0 = slot * _RPT

    pltpu.sync_copy(a1_hbm.at[b], xv)
    pltpu.sync_copy(a2_hbm.at[b], yv)

    def prep_y(k, carry):
        s = k * 16
        v0 = yv[0, pl.ds(s, 16)]
        v1 = yv[1, pl.ds(s, 16)]
        v2 = yv[2, pl.ds(s, 16)]
        yr[0, pl.ds(s, 16)] = _round_bf16(v0)
        yr[1, pl.ds(s, 16)] = _round_bf16(v1)
        yr[2, pl.ds(s, 16)] = _round_bf16(v2)
        y2b[pl.ds(s, 16)] = v0 * v0 + v1 * v1 + v2 * v2
        colp[pl.ds(s, 16)] = _INF16()
        return carry

    lax.fori_loop(0, _N // 16, prep_y, 0)

    def prep_x(k, carry):
        s = k * 16
        v0 = xv[0, pl.ds(r0 + s, 16)]
        v1 = xv[1, pl.ds(r0 + s, 16)]
        v2 = xv[2, pl.ds(r0 + s, 16)]
        xr[0, pl.ds(s, 16)] = _round_bf16(v0)
        xr[1, pl.ds(s, 16)] = _round_bf16(v1)
        xr[2, pl.ds(s, 16)] = _round_bf16(v2)
        x2b[pl.ds(s, 16)] = v0 * v0 + v1 * v1 + v2 * v2
        return carry

    lax.fori_loop(0, _RPT // 16, prep_x, 0)

    lanes = lax.iota(jnp.int32, 16)
    jconsts = [jnp.full((16,), j, jnp.int32) for j in range(16)]
    xors = [lanes ^ jnp.int32(k) for k in (8, 4, 2, 1)]
    masks = [lanes == jnp.int32(j) for j in range(16)]

    rowsum = jnp.zeros((16,), jnp.float32)
    for half in range(_HALVES):
        xbase = half * 16 * _G
        xs = []
        x2s = []
        for gi in range(_G):
            off = xbase + gi * 16
            xs.append((xr[0, pl.ds(off, 16)],
                       xr[1, pl.ds(off, 16)],
                       xr[2, pl.ds(off, 16)]))
            x2s.append(x2b[pl.ds(off, 16)])

        def mbody(mc, accs, xs=xs, x2s=x2s):
            s = mc * 16
            o0 = yr[0, pl.ds(s, 16)]
            o0 = o0 + o0
            o1 = yr[1, pl.ds(s, 16)]
            o1 = o1 + o1
            o2 = yr[2, pl.ds(s, 16)]
            o2 = o2 + o2
            oy2 = y2b[pl.ds(s, 16)]
            accs = list(accs)
            colv = _INF16()
            for j in range(16):
                b0 = _take16(o0, jconsts[j])
                b1 = _take16(o1, jconsts[j])
                b2 = _take16(o2, jconsts[j])
                y2v = _take16(oy2, jconsts[j])
                parts = []
                for gi in range(_G):
                    q0, q1, q2 = xs[gi]
                    t = q0 * b0 + q1 * b1 + q2 * b2
                    dd = (x2s[gi] + y2v) - t
                    dd = jnp.maximum(dd, jnp.float32(0.0))
                    accs[gi] = jnp.minimum(accs[gi], dd)
                    parts.append(dd)
                p = _tree_min(parts)
                for xk in xors:
                    p = jnp.minimum(p, _take16(p, xk))
                colv = jnp.where(masks[j], p, colv)
            colp[pl.ds(s, 16)] = jnp.minimum(colp[pl.ds(s, 16)], colv)
            return tuple(accs)

        accs = lax.fori_loop(0, _N // 16, mbody,
                             tuple(_INF16() for _ in range(_G)))
        for gi in range(_G):
            rowsum = rowsum + _sqrt16(accs[gi])

    pltpu.sync_copy(colp, shared.at[sid])
    plsc.subcore_barrier()
    sbase = (sid // 8) * 8
    m0 = slot * _RPT
    pltpu.sync_copy(shared.at[pl.ds(sbase, 8), pl.ds(m0, _RPT)], tmp8)

    colsum = jnp.zeros((16,), jnp.float32)
    for k in range(_RPT // 16):
        vs = [tmp8[p, pl.ds(k * 16, 16)] for p in range(8)]
        colsum = colsum + _sqrt16(_tree_min(vs))

    accv[0, :] = rowsum
    accv[1, :] = colsum
    pltpu.sync_copy(accv, out_hbm.at[cid * 16 + sid])


@jax.jit
def _sc_chamfer(a1t, a2t):
    mesh = plsc.VectorSubcoreMesh(core_axis_name="c", subcore_axis_name="s")
    run = pl.kernel(
        _sc_body,
        out_type=jax.ShapeDtypeStruct((32, 2, 16), jnp.float32),
        mesh=mesh,
        scratch_types=[
            pltpu.VMEM((3, _N), jnp.float32),
            pltpu.VMEM((3, _N), jnp.float32),
            pltpu.VMEM((3, _N), jnp.float32),
            pltpu.VMEM((_N,), jnp.float32),
            pltpu.VMEM((3, _RPT), jnp.float32),
            pltpu.VMEM((_RPT,), jnp.float32),
            pltpu.VMEM((_N,), jnp.float32),
            pltpu.VMEM((8, _RPT), jnp.float32),
            pltpu.VMEM((2, 16), jnp.float32),
            pltpu.VMEM_SHARED((16, _N), jnp.float32),
        ],
    )
    return run(a1t, a2t)


def kernel(array1, array2):
    a1t = jnp.transpose(array1, (0, 2, 1))
    a2t = jnp.transpose(array2, (0, 2, 1))
    parts = _sc_chamfer(a1t, a2t)
    total = jnp.sum(parts)
    return total / jnp.float32(2 * _B * _N)
